# Initial kernel scaffold; baseline (speedup 1.0000x reference)
import jax
import jax.numpy as jnp
from jax import lax
from jax.experimental import pallas as pl
from jax.experimental.pallas import tpu as pltpu
from jax.experimental.pallas import tpu_sc as plsc

_NB = 64
_GRID = _NB * _NB * _NB
_B = 16
_P = 131072
_NC = 2
_NS = 16
_L = 16
_ROUNDS = _B // _NC
_PPT = _P // _NS
_FPT = _PPT * 3
_NVEC = _FPT // _L
_GSLICE = _GRID // _NS
_CHUNK = 128
_NCHUNK = _PPT // _CHUNK


def _body(x_hbm, out_hbm, pts, idx, ones, zeros, maxv, allmax,
          shmax, grid_sh, sem):
    c = lax.axis_index("c")
    s = lax.axis_index("s")
    lane = lax.iota(jnp.int32, _L)

    for k in range(_CHUNK // _L):
        ones[pl.ds(k * _L, _L)] = jnp.ones((_L,), jnp.float32)

    def zero_body(i, _):
        zeros[pl.ds(i * _L, _L)] = jnp.zeros((_L,), jnp.float32)
        return 0
    lax.fori_loop(0, _GSLICE // _L, zero_body, 0)

    pltpu.sync_copy(zeros, grid_sh.at[pl.ds(s * _GSLICE, _GSLICE)])

    def round_body(r, _):
        b = c * _ROUNDS + r

        pltpu.sync_copy(x_hbm.at[b, pl.ds(s * _FPT, _FPT)], pts)

        def max_body(i, m):
            v = pts[pl.ds(i * _L, _L)]
            return jnp.maximum(m, jnp.abs(v))
        m = lax.fori_loop(0, _NVEC, max_body, jnp.zeros((_L,), jnp.float32))
        maxv[...] = m
        pltpu.sync_copy(maxv, shmax.at[s])
        plsc.subcore_barrier()

        pltpu.sync_copy(shmax, allmax)

        def gmax_body(i, mm):
            return jnp.maximum(mm, allmax[i])
        mm = lax.fori_loop(0, _NS, gmax_body, jnp.zeros((_L,), jnp.float32))
        gmax = jnp.max(mm)
        thick = jnp.maximum(2.0 * gmax, 1e-5)

        def chunk_body(g, _):
            for v in range(_CHUNK // _L):
                pid = lane + (g * _CHUNK + v * _L)
                fx = pid * 3
                x = plsc.load_gather(pts, [fx])
                y = plsc.load_gather(pts, [fx + 1])
                z = plsc.load_gather(pts, [fx + 2])
                cx = ((x + gmax) / thick * 64.0).astype(jnp.int32)
                cy = ((y + gmax) / thick * 64.0).astype(jnp.int32)
                cz = ((z + gmax) / thick * 64.0).astype(jnp.int32)
                cx = jnp.clip(cx, 0, _NB - 1)
                cy = jnp.clip(cy, 0, _NB - 1)
                cz = jnp.clip(cz, 0, _NB - 1)
                flat = (cx * _NB + cy) * _NB + cz
                idx[g, pl.ds(v * _L, _L)] = flat
            pltpu.async_copy(ones, grid_sh.at[idx.at[g]], sem)
            return 0
        lax.fori_loop(0, _NCHUNK, chunk_body, 0)

        def drain_body(g, _):
            pltpu.make_async_copy(ones, grid_sh.at[idx.at[g]], sem).wait()
            return 0
        lax.fori_loop(0, _NCHUNK, drain_body, 0)
        plsc.subcore_barrier()

        sl = pl.ds(s * _GSLICE, _GSLICE)
        pltpu.sync_copy(grid_sh.at[sl], out_hbm.at[b, sl])
        pltpu.sync_copy(zeros, grid_sh.at[sl])
        return 0

    lax.fori_loop(0, _ROUNDS, round_body, 0)


_occupancy = pl.kernel(
    _body,
    out_type=jax.ShapeDtypeStruct((_B, _GRID), jnp.float32),
    mesh=plsc.VectorSubcoreMesh(
        core_axis_name="c", subcore_axis_name="s",
        num_cores=_NC, num_subcores=_NS),
    scratch_types=[
        pltpu.VMEM((_FPT,), jnp.float32),
        pltpu.VMEM((_NCHUNK, _CHUNK), jnp.int32),
        pltpu.VMEM((_CHUNK,), jnp.float32),
        pltpu.VMEM((_GSLICE,), jnp.float32),
        pltpu.VMEM((_L,), jnp.float32),
        pltpu.VMEM((_NS, _L), jnp.float32),
        pltpu.VMEM_SHARED((_NS, _L), jnp.float32),
        pltpu.VMEM_SHARED((_GRID,), jnp.float32),
        pltpu.SemaphoreType.DMA,
    ],
)


def kernel(input):
    x = input.reshape(_B, _P * 3)
    return _occupancy(x)


# speedup vs baseline: 4.4561x; 4.4561x over previous
import jax
import jax.numpy as jnp
from jax import lax
from jax.experimental import pallas as pl
from jax.experimental.pallas import tpu as pltpu
from jax.experimental.pallas import tpu_sc as plsc

_NB = 64
_GRID = _NB * _NB * _NB
_B = 16
_P = 131072
_NC = 2
_NS = 16
_L = 16
_ROUNDS = _B // _NC
_PPT = _P // _NS
_FPT = _PPT * 3
_NVEC = _FPT // _L
_GSLICE = _GRID // _NS
_CHUNK = 128
_NCHUNK = _PPT // _CHUNK
_RING = 4


def _body(x_hbm, out_hbm, pts, idx0, idx1, idx2, idx3, ones, zeros,
          maxv, allmax, shared, sem):
    idxs = (idx0, idx1, idx2, idx3)
    c = lax.axis_index("c")
    s = lax.axis_index("s")
    lane = lax.iota(jnp.int32, _L)

    for k in range(_CHUNK // _L):
        ones[pl.ds(k * _L, _L)] = jnp.ones((_L,), jnp.float32)

    def zero_body(i, _):
        zeros[pl.ds(i * _L, _L)] = jnp.zeros((_L,), jnp.float32)
        return 0
    lax.fori_loop(0, _GSLICE // _L, zero_body, 0)

    pltpu.sync_copy(zeros, shared.at[pl.ds(s * _GSLICE, _GSLICE)])

    def round_body(r, _):
        b = c * _ROUNDS + r

        pltpu.sync_copy(x_hbm.at[b, pl.ds(s * _FPT, _FPT)], pts)

        def max_body(i, m):
            v = pts[pl.ds(i * _L, _L)]
            return jnp.maximum(m, jnp.abs(v))
        m = lax.fori_loop(0, _NVEC, max_body, jnp.zeros((_L,), jnp.float32))
        maxv[...] = m
        pltpu.sync_copy(maxv, shared.at[pl.ds(_GRID + s * _L, _L)])
        plsc.subcore_barrier()

        pltpu.sync_copy(shared.at[pl.ds(_GRID, _NS * _L)], allmax)

        def gmax_body(i, mm):
            return jnp.maximum(mm, allmax[pl.ds(i * _L, _L)])
        mm = lax.fori_loop(0, _NS, gmax_body, jnp.zeros((_L,), jnp.float32))
        gmax = mm[0]
        for i in range(1, _L):
            gmax = jnp.maximum(gmax, mm[i])
        thick = jnp.maximum(2.0 * gmax, 1e-5)

        def super_body(go, _):
            for j in range(_RING):
                g = go * _RING + j
                for v in range(_CHUNK // _L):
                    pid = lane + (g * _CHUNK + v * _L)
                    fx = pid * 3
                    x = plsc.load_gather(pts, [fx])
                    y = plsc.load_gather(pts, [fx + 1])
                    z = plsc.load_gather(pts, [fx + 2])
                    cx = ((x + gmax) / thick * 64.0).astype(jnp.int32)
                    cy = ((y + gmax) / thick * 64.0).astype(jnp.int32)
                    cz = ((z + gmax) / thick * 64.0).astype(jnp.int32)
                    cx = jnp.clip(cx, 0, _NB - 1)
                    cy = jnp.clip(cy, 0, _NB - 1)
                    cz = jnp.clip(cz, 0, _NB - 1)
                    flat = (cx * _NB + cy) * _NB + cz
                    idxs[j][pl.ds(v * _L, _L)] = flat
                pltpu.async_copy(ones, shared.at[idxs[j]], sem)
            for j in range(_RING):
                pltpu.make_async_copy(ones, shared.at[idxs[j]], sem).wait()
            return 0
        lax.fori_loop(0, _NCHUNK // _RING, super_body, 0)
        plsc.subcore_barrier()

        sl = pl.ds(s * _GSLICE, _GSLICE)
        pltpu.sync_copy(shared.at[sl], out_hbm.at[b, sl])
        pltpu.sync_copy(zeros, shared.at[sl])
        return 0

    lax.fori_loop(0, _ROUNDS, round_body, 0)


_occupancy = pl.kernel(
    _body,
    out_type=jax.ShapeDtypeStruct((_B, _GRID), jnp.float32),
    mesh=plsc.VectorSubcoreMesh(
        core_axis_name="c", subcore_axis_name="s",
        num_cores=_NC, num_subcores=_NS),
    compiler_params=pltpu.CompilerParams(needs_layout_passes=False),
    scratch_types=[
        pltpu.VMEM((_FPT,), jnp.float32),
        pltpu.VMEM((_CHUNK,), jnp.int32),
        pltpu.VMEM((_CHUNK,), jnp.int32),
        pltpu.VMEM((_CHUNK,), jnp.int32),
        pltpu.VMEM((_CHUNK,), jnp.int32),
        pltpu.VMEM((_CHUNK,), jnp.float32),
        pltpu.VMEM((_GSLICE,), jnp.float32),
        pltpu.VMEM((_L,), jnp.float32),
        pltpu.VMEM((_NS * _L,), jnp.float32),
        pltpu.VMEM_SHARED((_GRID + _NS * _L,), jnp.float32),
        pltpu.SemaphoreType.DMA,
    ],
)


def kernel(input):
    x = input.reshape(_B, _P * 3)
    return _occupancy(x)
